# Initial kernel scaffold; baseline (speedup 1.0000x reference)
import jax
import jax.numpy as jnp
from jax.experimental import pallas as pl


def kernel(x, edge_indices, W0, a_src0, a_dst0, b0, W1, a_src1, a_dst1, b1, W2, a_src2, a_dst2, b2):
    raise NotImplementedError("write your pallas kernel here")


# speedup vs baseline: 31.4672x; 31.4672x over previous
import functools

import jax
import jax.numpy as jnp
from jax import lax
from jax.experimental import pallas as pl
from jax.experimental.pallas import tpu as pltpu
from jax.experimental.pallas import tpu_sc as plsc

N = 10000
E = 320000
HID = 128
HEADS = 4
OUT_CH = HID // HEADS
ACC_W = 144
ADT_W = 16
NEG_SLOPE = 0.2
EPS = 1e-16

NC = 2
NS = 16
NW = NC * NS
EPW = E // NW
CH = 80
NCHUNK = EPW // CH
ROWS_PT = N // NS


_mesh = plsc.VectorSubcoreMesh(core_axis_name="c", subcore_axis_name="s")


@functools.partial(
    pl.kernel,
    out_type=pltpu.HBM((NC, N, ACC_W), jnp.float32),
    mesh=_mesh,
    scratch_types=[
        pltpu.VMEM((CH,), jnp.int32),
        pltpu.VMEM((CH,), jnp.int32),
        pltpu.VMEM((CH, ACC_W), jnp.float32),
        pltpu.VMEM((CH, ADT_W), jnp.float32),
        pltpu.VMEM((CH, ACC_W), jnp.float32),
        pltpu.VMEM_SHARED((N, ACC_W), jnp.float32),
        pltpu.SemaphoreType.DMA,
        pltpu.SemaphoreType.DMA,
    ],
    compiler_params=pltpu.CompilerParams(
        use_tc_tiling_on_sc=False, needs_layout_passes=False),
)
def _gat_edge_pass(ha_hbm, adt_hbm, src_hbm, dst_hbm, out_hbm,
                   srcv, dstv, rowsa, adg, stage, accum, gsem, asem):
    cid = lax.axis_index("c")
    sid = lax.axis_index("s")
    wid = sid * NC + cid

    zero16 = jnp.zeros((16,), jnp.float32)

    @pl.loop(0, CH)
    def _zero_stage(e):
        for v in range(ACC_W // 16):
            stage[e, pl.ds(v * 16, 16)] = zero16

    for j in range(N // CH):
        @pl.when(sid == j % NS)
        def _zero_accum():
            pltpu.sync_copy(stage, accum.at[pl.ds(j * CH, CH)])

    plsc.subcore_barrier()

    @pl.loop(0, NCHUNK)
    def _chunk(k):
        ebase = wid * EPW + k * CH
        pltpu.sync_copy(src_hbm.at[pl.ds(ebase, CH)], srcv)
        pltpu.sync_copy(dst_hbm.at[pl.ds(ebase, CH)], dstv)
        gat = pltpu.async_copy(ha_hbm.at[srcv], rowsa, gsem)
        adt = pltpu.async_copy(adt_hbm.at[dstv], adg, asem)
        adt.wait()
        gat.wait()

        for g in range(CH // 16):
            rid = lax.iota(jnp.int32, 16) + (g * 16)
            for hd in range(HEADS):
                a = (plsc.load_gather(rowsa, [rid, jnp.full((16,), HID + hd, jnp.int32)])
                     + plsc.load_gather(adg, [rid, jnp.full((16,), hd, jnp.int32)]))
                a = jnp.maximum(a, a * NEG_SLOPE)
                ex = jnp.exp(a)
                plsc.store_scatter(
                    stage, [rid, jnp.full((16,), HID + hd, jnp.int32)], ex)

        @pl.loop(0, CH)
        def _scale(e):
            erow = jnp.zeros((16,), jnp.int32) + e
            for hd in range(HEADS):
                m = plsc.load_gather(
                    stage, [erow, jnp.full((16,), HID + hd, jnp.int32)])
                for j in range(2):
                    c0 = (hd * 2 + j) * 16
                    stage[e, pl.ds(c0, 16)] = rowsa[e, pl.ds(c0, 16)] * m

        pltpu.sync_copy(stage, accum.at[dstv], add=True)

    plsc.subcore_barrier()
    for j in range(N // CH):
        @pl.when(sid == j % NS)
        def _copy_out():
            pltpu.sync_copy(accum.at[pl.ds(j * CH, CH)], stage)
            pltpu.sync_copy(stage, out_hbm.at[cid, pl.ds(j * CH, CH)])


TCB = 2000


def _write_packed(h, avs_ref, avd_ref, ha_ref, adt_ref):
    ha_ref[:, 0:HID] = h
    zcol = jnp.zeros((TCB, 1), jnp.float32)
    for hd in range(HEADS):
        sl = slice(hd * OUT_CH, (hd + 1) * OUT_CH)
        ha_ref[:, HID + hd:HID + hd + 1] = jnp.sum(
            h[:, sl] * avs_ref[:, sl], axis=1, keepdims=True)
        adt_ref[:, hd:hd + 1] = jnp.sum(
            h[:, sl] * avd_ref[:, sl], axis=1, keepdims=True)
    for c in range(HID + HEADS, ACC_W):
        ha_ref[:, c:c + 1] = zcol
    for c in range(HEADS, ADT_W):
        adt_ref[:, c:c + 1] = zcol


def _tc_first_body(x_ref, w_ref, avs_ref, avd_ref, ha_ref, adt_ref):
    h = jnp.dot(x_ref[...], w_ref[...], preferred_element_type=jnp.float32,
                precision=lax.Precision.HIGHEST)
    _write_packed(h, avs_ref, avd_ref, ha_ref, adt_ref)


def _tc_mid_body(p_ref, b_ref, w_ref, avs_ref, avd_ref, ha_ref, adt_ref):
    p = p_ref[0] + p_ref[1]
    hin = jnp.concatenate(
        [jnp.maximum(p[:, hd * OUT_CH:(hd + 1) * OUT_CH]
                     / (p[:, HID + hd:HID + hd + 1] + EPS)
                     + b_ref[:, hd * OUT_CH:(hd + 1) * OUT_CH], 0.0)
         for hd in range(HEADS)], axis=1)
    h = jnp.dot(hin, w_ref[...], preferred_element_type=jnp.float32,
                precision=lax.Precision.HIGHEST)
    _write_packed(h, avs_ref, avd_ref, ha_ref, adt_ref)


def _tc_final_body(p_ref, b_ref, o_ref):
    p = p_ref[0] + p_ref[1]
    for hd in range(HEADS):
        sl = slice(hd * OUT_CH, (hd + 1) * OUT_CH)
        den = p[:, HID + hd:HID + hd + 1] + EPS
        o_ref[:, sl] = jnp.maximum(p[:, sl] / den + b_ref[:, sl], 0.0)


_PACKED_OUT = (jax.ShapeDtypeStruct((N, ACC_W), jnp.float32),
               jax.ShapeDtypeStruct((N, ADT_W), jnp.float32))
_PACKED_SPECS = [pl.BlockSpec((TCB, ACC_W), lambda i: (i, 0)),
                 pl.BlockSpec((TCB, ADT_W), lambda i: (i, 0))]
_P_SPEC = pl.BlockSpec((2, TCB, ACC_W), lambda i: (0, i, 0))
_ROW_SPEC = pl.BlockSpec((1, HID), lambda i: (0, 0))
_W_SPEC = pl.BlockSpec((HID, HID), lambda i: (0, 0))


def _tc_first(x, w, avs, avd):
    return pl.pallas_call(
        _tc_first_body,
        grid=(N // TCB,),
        in_specs=[pl.BlockSpec((TCB, HID), lambda i: (i, 0)),
                  _W_SPEC, _ROW_SPEC, _ROW_SPEC],
        out_specs=_PACKED_SPECS,
        out_shape=_PACKED_OUT,
    )(x, w, avs, avd)


def _tc_mid(p, b, w, avs, avd):
    return pl.pallas_call(
        _tc_mid_body,
        grid=(N // TCB,),
        in_specs=[_P_SPEC, _ROW_SPEC, _W_SPEC, _ROW_SPEC, _ROW_SPEC],
        out_specs=_PACKED_SPECS,
        out_shape=_PACKED_OUT,
    )(p, b, w, avs, avd)


def _tc_final(p, b):
    return pl.pallas_call(
        _tc_final_body,
        grid=(N // TCB,),
        in_specs=[_P_SPEC, _ROW_SPEC],
        out_specs=pl.BlockSpec((TCB, HID), lambda i: (i, 0)),
        out_shape=jax.ShapeDtypeStruct((N, HID), jnp.float32),
    )(p, b)


def kernel(x, edge_indices, W0, a_src0, a_dst0, b0,
           W1, a_src1, a_dst1, b1, W2, a_src2, a_dst2, b2):
    Ws = (W0, W1, W2)
    avs = tuple(a.reshape(1, HID) for a in (a_src0, a_src1, a_src2))
    avd = tuple(a.reshape(1, HID) for a in (a_dst0, a_dst1, a_dst2))
    bs = tuple(b.reshape(1, HID) for b in (b0, b1, b2))

    ha, adt = _tc_first(x, Ws[0], avs[0], avd[0])
    part = None
    for i in range(3):
        src = edge_indices[i, 0].astype(jnp.int32)
        dst = edge_indices[i, 1].astype(jnp.int32)
        part = _gat_edge_pass(ha, adt, src, dst)
        if i < 2:
            ha, adt = _tc_mid(part, bs[i], Ws[i + 1], avs[i + 1], avd[i + 1])
    return _tc_final(part, bs[2])


# speedup vs baseline: 51.6079x; 1.6401x over previous
import functools

import jax
import jax.numpy as jnp
from jax import lax
from jax.experimental import pallas as pl
from jax.experimental.pallas import tpu as pltpu
from jax.experimental.pallas import tpu_sc as plsc

N = 10000
E = 320000
HID = 128
HEADS = 4
OUT_CH = HID // HEADS
ACC_W = 144
ADT_W = 16
NEG_SLOPE = 0.2
EPS = 1e-16

NC = 2
NS = 16
NW = NC * NS
EPW = E // NW
CH = 80
NCHUNK = EPW // CH
ROWS_PT = N // NS


_mesh = plsc.VectorSubcoreMesh(core_axis_name="c", subcore_axis_name="s")


@functools.partial(
    pl.kernel,
    out_type=pltpu.HBM((NC, N, ACC_W), jnp.float32),
    mesh=_mesh,
    scratch_types=[
        pltpu.VMEM((2, CH), jnp.int32),
        pltpu.VMEM((2, CH), jnp.int32),
        pltpu.VMEM((CH, ACC_W), jnp.float32),
        pltpu.VMEM((CH, ACC_W), jnp.float32),
        pltpu.VMEM((CH, ADT_W), jnp.float32),
        pltpu.VMEM((CH, ADT_W), jnp.float32),
        pltpu.VMEM_SHARED((N, ACC_W), jnp.float32),
        pltpu.SemaphoreType.DMA,
        pltpu.SemaphoreType.DMA,
        pltpu.SemaphoreType.DMA,
        pltpu.SemaphoreType.DMA,
    ],
    compiler_params=pltpu.CompilerParams(
        use_tc_tiling_on_sc=False, needs_layout_passes=False),
)
def _gat_edge_pass(ha_hbm, adt_hbm, src_hbm, dst_hbm, out_hbm,
                   idxa, idxb, rowsa, rowsb, adga, adgb,
                   accum, ga, gb, sa, sb):
    cid = lax.axis_index("c")
    sid = lax.axis_index("s")
    wid = sid * NC + cid

    zero16 = jnp.zeros((16,), jnp.float32)

    @pl.loop(0, CH)
    def _zero_stage(e):
        for v in range(ACC_W // 16):
            rowsa[e, pl.ds(v * 16, 16)] = zero16

    for j in range(N // CH):
        @pl.when(sid == j % NS)
        def _zero_accum():
            pltpu.sync_copy(rowsa, accum.at[pl.ds(j * CH, CH)])

    plsc.subcore_barrier()

    def _prefetch(k, idx2, rows_, adg_, gs):
        ebase = wid * EPW + k * CH
        pltpu.sync_copy(src_hbm.at[pl.ds(ebase, CH)], idx2.at[0])
        pltpu.sync_copy(dst_hbm.at[pl.ds(ebase, CH)], idx2.at[1])
        pltpu.async_copy(ha_hbm.at[idx2.at[0]], rows_, gs)
        pltpu.async_copy(adt_hbm.at[idx2.at[1]], adg_, gs)

    def _wait_gathers(idx2, rows_, adg_, gs):
        pltpu.make_async_copy(ha_hbm.at[idx2.at[0]], rows_, gs).wait()
        pltpu.make_async_copy(adt_hbm.at[idx2.at[1]], adg_, gs).wait()

    def _compute(rows_, adg_):
        for g in range(CH // 16):
            rid = lax.iota(jnp.int32, 16) + (g * 16)
            for hd in range(HEADS):
                a = (plsc.load_gather(rows_, [rid, jnp.full((16,), HID + hd, jnp.int32)])
                     + plsc.load_gather(adg_, [rid, jnp.full((16,), hd, jnp.int32)]))
                a = jnp.maximum(a, a * NEG_SLOPE)
                ex = jnp.exp(a)
                plsc.store_scatter(
                    rows_, [rid, jnp.full((16,), HID + hd, jnp.int32)], ex)

        @pl.loop(0, CH, unroll=2)
        def _scale(e):
            erow = jnp.zeros((16,), jnp.int32) + e
            for hd in range(HEADS):
                m = plsc.load_gather(
                    rows_, [erow, jnp.full((16,), HID + hd, jnp.int32)])
                for j in range(2):
                    c0 = (hd * 2 + j) * 16
                    rows_[e, pl.ds(c0, 16)] = rows_[e, pl.ds(c0, 16)] * m

    def _scatter(idx2, rows_, ss):
        pltpu.async_copy(rows_, accum.at[idx2.at[1]], ss, add=True)

    def _wait_scatter(idx2, rows_, ss):
        pltpu.make_async_copy(rows_, accum.at[idx2.at[1]], ss).wait()

    _prefetch(0, idxa, rowsa, adga, ga)
    _prefetch(1, idxb, rowsb, adgb, gb)

    @pl.loop(0, NCHUNK // 2)
    def _pair(m):
        ka = 2 * m
        _wait_gathers(idxa, rowsa, adga, ga)
        _compute(rowsa, adga)
        _scatter(idxa, rowsa, sa)
        _wait_gathers(idxb, rowsb, adgb, gb)
        _compute(rowsb, adgb)
        _scatter(idxb, rowsb, sb)
        _wait_scatter(idxa, rowsa, sa)
        _prefetch(ka + 2, idxa, rowsa, adga, ga)
        knb = ka + 3
        _wait_scatter(idxb, rowsb, sb)
        _prefetch(jnp.where(knb >= NCHUNK, 0, knb), idxb, rowsb, adgb, gb)

    _wait_gathers(idxa, rowsa, adga, ga)
    _compute(rowsa, adga)
    _scatter(idxa, rowsa, sa)
    _wait_gathers(idxb, rowsb, adgb, gb)
    _wait_scatter(idxa, rowsa, sa)

    plsc.subcore_barrier()
    for j in range(N // CH):
        @pl.when(sid == j % NS)
        def _copy_out():
            pltpu.sync_copy(accum.at[pl.ds(j * CH, CH)], rowsa)
            pltpu.sync_copy(rowsa, out_hbm.at[cid, pl.ds(j * CH, CH)])


TCB = 2000


def _write_packed(h, avs_ref, avd_ref, ha_ref, adt_ref):
    ha_ref[:, 0:HID] = h
    zcol = jnp.zeros((TCB, 1), jnp.float32)
    for hd in range(HEADS):
        sl = slice(hd * OUT_CH, (hd + 1) * OUT_CH)
        ha_ref[:, HID + hd:HID + hd + 1] = jnp.sum(
            h[:, sl] * avs_ref[:, sl], axis=1, keepdims=True)
        adt_ref[:, hd:hd + 1] = jnp.sum(
            h[:, sl] * avd_ref[:, sl], axis=1, keepdims=True)
    for c in range(HID + HEADS, ACC_W):
        ha_ref[:, c:c + 1] = zcol
    for c in range(HEADS, ADT_W):
        adt_ref[:, c:c + 1] = zcol


def _tc_first_body(x_ref, w_ref, avs_ref, avd_ref, ha_ref, adt_ref):
    h = jnp.dot(x_ref[...], w_ref[...], preferred_element_type=jnp.float32,
                precision=lax.Precision.HIGHEST)
    _write_packed(h, avs_ref, avd_ref, ha_ref, adt_ref)


def _tc_mid_body(p_ref, b_ref, w_ref, avs_ref, avd_ref, ha_ref, adt_ref):
    p = p_ref[0] + p_ref[1]
    hin = jnp.concatenate(
        [jnp.maximum(p[:, hd * OUT_CH:(hd + 1) * OUT_CH]
                     / (p[:, HID + hd:HID + hd + 1] + EPS)
                     + b_ref[:, hd * OUT_CH:(hd + 1) * OUT_CH], 0.0)
         for hd in range(HEADS)], axis=1)
    h = jnp.dot(hin, w_ref[...], preferred_element_type=jnp.float32,
                precision=lax.Precision.HIGHEST)
    _write_packed(h, avs_ref, avd_ref, ha_ref, adt_ref)


def _tc_final_body(p_ref, b_ref, o_ref):
    p = p_ref[0] + p_ref[1]
    for hd in range(HEADS):
        sl = slice(hd * OUT_CH, (hd + 1) * OUT_CH)
        den = p[:, HID + hd:HID + hd + 1] + EPS
        o_ref[:, sl] = jnp.maximum(p[:, sl] / den + b_ref[:, sl], 0.0)


_PACKED_OUT = (jax.ShapeDtypeStruct((N, ACC_W), jnp.float32),
               jax.ShapeDtypeStruct((N, ADT_W), jnp.float32))
_PACKED_SPECS = [pl.BlockSpec((TCB, ACC_W), lambda i: (i, 0)),
                 pl.BlockSpec((TCB, ADT_W), lambda i: (i, 0))]
_P_SPEC = pl.BlockSpec((2, TCB, ACC_W), lambda i: (0, i, 0))
_ROW_SPEC = pl.BlockSpec((1, HID), lambda i: (0, 0))
_W_SPEC = pl.BlockSpec((HID, HID), lambda i: (0, 0))


def _tc_first(x, w, avs, avd):
    return pl.pallas_call(
        _tc_first_body,
        grid=(N // TCB,),
        in_specs=[pl.BlockSpec((TCB, HID), lambda i: (i, 0)),
                  _W_SPEC, _ROW_SPEC, _ROW_SPEC],
        out_specs=_PACKED_SPECS,
        out_shape=_PACKED_OUT,
    )(x, w, avs, avd)


def _tc_mid(p, b, w, avs, avd):
    return pl.pallas_call(
        _tc_mid_body,
        grid=(N // TCB,),
        in_specs=[_P_SPEC, _ROW_SPEC, _W_SPEC, _ROW_SPEC, _ROW_SPEC],
        out_specs=_PACKED_SPECS,
        out_shape=_PACKED_OUT,
    )(p, b, w, avs, avd)


def _tc_final(p, b):
    return pl.pallas_call(
        _tc_final_body,
        grid=(N // TCB,),
        in_specs=[_P_SPEC, _ROW_SPEC],
        out_specs=pl.BlockSpec((TCB, HID), lambda i: (i, 0)),
        out_shape=jax.ShapeDtypeStruct((N, HID), jnp.float32),
    )(p, b)


def kernel(x, edge_indices, W0, a_src0, a_dst0, b0,
           W1, a_src1, a_dst1, b1, W2, a_src2, a_dst2, b2):
    Ws = (W0, W1, W2)
    avs = tuple(a.reshape(1, HID) for a in (a_src0, a_src1, a_src2))
    avd = tuple(a.reshape(1, HID) for a in (a_dst0, a_dst1, a_dst2))
    bs = tuple(b.reshape(1, HID) for b in (b0, b1, b2))

    ha, adt = _tc_first(x, Ws[0], avs[0], avd[0])
    part = None
    for i in range(3):
        src = edge_indices[i, 0].astype(jnp.int32)
        dst = edge_indices[i, 1].astype(jnp.int32)
        part = _gat_edge_pass(ha, adt, src, dst)
        if i < 2:
            ha, adt = _tc_mid(part, bs[i], Ws[i + 1], avs[i + 1], avd[i + 1])
    return _tc_final(part, bs[2])
